# baseline (speedup 1.0000x reference)
import jax
import jax.numpy as jnp
from jax import lax
from jax.experimental import pallas as pl
from jax.experimental.pallas import tpu as pltpu
from jax.experimental.pallas import tpu_sc as plsc

TM = 1024
NC, NS, L = 2, 16, 16
NW = NC * NS
TOKENS = 16384
NEXP = 64
RPW = TOKENS // NW
NGRP = RPW // L


def _matmul_block(x_ref, wt_ref, s_ref):
    s_ref[...] = jax.lax.dot_general(
        x_ref[...], wt_ref[...], (((1,), (0,)), ((), ())),
        preferred_element_type=jnp.float32)


def _scores(x, wt):
    tokens, d = x.shape
    n_exp = wt.shape[1]
    return pl.pallas_call(
        _matmul_block,
        grid=(tokens // TM,),
        in_specs=[
            pl.BlockSpec((TM, d), lambda i: (i, 0)),
            pl.BlockSpec((d, n_exp), lambda i: (0, 0)),
        ],
        out_specs=pl.BlockSpec((TM, n_exp), lambda i: (i, 0)),
        out_shape=jax.ShapeDtypeStruct((tokens, n_exp), jnp.float32),
    )(x, wt)


def _sc_top2(s_hbm, topi_hbm, topv_hbm, sv, iv, vv):
    wid = lax.axis_index("s") * NC + lax.axis_index("c")
    base = wid * RPW
    pltpu.sync_copy(s_hbm.at[pl.ds(base * NEXP, RPW * NEXP)], sv)

    lanes = lax.iota(jnp.int32, L)

    def group(g, carry):
        rows = g * L + lanes
        neg = jnp.full((L,), -jnp.inf, jnp.float32)
        m1, m2 = neg, neg
        i1 = jnp.zeros((L,), jnp.int32)
        i2 = i1
        flat = rows * NEXP
        for e in range(NEXP):
            ev = jnp.full((L,), e, jnp.int32)
            v = plsc.load_gather(sv, [flat + e])
            gt1 = v > m1
            gt2 = v > m2
            nm2 = jnp.where(gt1, m1, jnp.where(gt2, v, m2))
            ni2 = jnp.where(gt1, i1, jnp.where(gt2, ev, i2))
            m1 = jnp.where(gt1, v, m1)
            i1 = jnp.where(gt1, ev, i1)
            m2, i2 = nm2, ni2
        e2 = jnp.exp(m2 - m1)
        inv = 1.0 / (1.0 + e2)
        out2 = rows * 2
        plsc.store_scatter(iv, [out2], i1)
        plsc.store_scatter(iv, [out2 + 1], i2)
        plsc.store_scatter(vv, [out2], inv)
        plsc.store_scatter(vv, [out2 + 1], e2 * inv)
        return carry

    lax.fori_loop(0, NGRP, group, 0)
    pltpu.sync_copy(iv, topi_hbm.at[pl.ds(base * 2, RPW * 2)])
    pltpu.sync_copy(vv, topv_hbm.at[pl.ds(base * 2, RPW * 2)])


_sc_call = pl.kernel(
    _sc_top2,
    out_type=(
        jax.ShapeDtypeStruct((TOKENS * 2,), jnp.int32),
        jax.ShapeDtypeStruct((TOKENS * 2,), jnp.float32),
    ),
    mesh=plsc.VectorSubcoreMesh(
        core_axis_name="c", subcore_axis_name="s",
        num_cores=NC, num_subcores=NS),
    scratch_types=[
        pltpu.VMEM((RPW * NEXP,), jnp.float32),
        pltpu.VMEM((RPW * 2,), jnp.int32),
        pltpu.VMEM((RPW * 2,), jnp.float32),
    ],
    compiler_params=pltpu.CompilerParams(needs_layout_passes=False),
)


@jax.jit
def kernel(x, W):
    scores = _scores(x, W.T)
    topi, topv = _sc_call(scores.reshape(-1))
    return (topi.reshape(TOKENS, 2), topv.reshape(TOKENS, 2))


# speedup vs baseline: 1.6261x; 1.6261x over previous
import jax
import jax.numpy as jnp
from jax.experimental import pallas as pl

TM = 1024


def _router_block(x_ref, w_ref, topi_ref, topv_ref):
    scores = jax.lax.dot_general(
        x_ref[...], w_ref[...], (((1,), (1,)), ((), ())),
        preferred_element_type=jnp.float32)
    e = scores.shape[1]
    iota = jax.lax.broadcasted_iota(jnp.int32, scores.shape, 1)

    m1 = jnp.max(scores, axis=1, keepdims=True)
    i1 = jnp.min(jnp.where(scores == m1, iota, e), axis=1, keepdims=True)
    masked = jnp.where(iota == i1, -jnp.inf, scores)
    m2 = jnp.max(masked, axis=1, keepdims=True)
    i2 = jnp.min(jnp.where(masked == m2, iota, e), axis=1, keepdims=True)

    z = jnp.sum(jnp.exp(scores - m1), axis=1, keepdims=True)
    e2 = jnp.exp(m2 - m1)
    inv = 1.0 / (1.0 + e2 + 1e-9 * z)
    topi_ref[...] = jnp.concatenate([i1, i2], axis=1)
    topv_ref[...] = jnp.concatenate([inv, e2 * inv], axis=1)


@jax.jit
def kernel(x, W):
    tokens, d = x.shape
    n_exp = W.shape[0]
    grid = (tokens // TM,)
    topi, topv = pl.pallas_call(
        _router_block,
        grid=grid,
        in_specs=[
            pl.BlockSpec((TM, d), lambda i: (i, 0)),
            pl.BlockSpec((n_exp, d), lambda i: (0, 0)),
        ],
        out_specs=[
            pl.BlockSpec((TM, 2), lambda i: (i, 0)),
            pl.BlockSpec((TM, 2), lambda i: (i, 0)),
        ],
        out_shape=[
            jax.ShapeDtypeStruct((tokens, 2), jnp.int32),
            jax.ShapeDtypeStruct((tokens, 2), jnp.float32),
        ],
    )(x, W)
    return (topi, topv)
